# baseline (speedup 1.0000x reference)
import functools

import jax
import jax.numpy as jnp
from jax import lax
from jax.experimental import pallas as pl
from jax.experimental.pallas import tpu as pltpu
from jax.experimental.pallas import tpu_sc as plsc

N = 10000
E = 160000
D = 256
Q = 64
TILES = 16
B = 128
NBLK = -(-E // (TILES * B))
EPT = NBLK * B
E_PAD = TILES * EPT
ACC_ROWS = 10240
ROWS_PT = ACC_ROWS // TILES


def _sc_body(xq0, xq1, xq2, xq3, src_hbm, dst_hbm,
             sq0, sq1, sq2, sq3, deg16_hbm,
             src_v, dst_v, bufs, deg_local,
             acc,
             sem_g, sem_s):
    c = lax.axis_index("c")
    s = lax.axis_index("s")

    pltpu.sync_copy(src_hbm.at[s], src_v)
    pltpu.sync_copy(dst_hbm.at[s], dst_v)

    r0 = s * ROWS_PT

    def zero_deg(i, carry):
        deg_local[pl.ds(i * 16, 16)] = jnp.zeros((16,), jnp.float32)
        return carry

    def phase(x_hbm, s_hbm, with_deg):
        if with_deg:
            lax.fori_loop(0, ACC_ROWS // 16, zero_deg, 0)

        pltpu.sync_copy(x_hbm.at[pl.ds(r0, ROWS_PT)], acc.at[pl.ds(r0, ROWS_PT)])
        plsc.subcore_barrier()

        def gather_start(j, p):
            pltpu.make_async_copy(x_hbm.at[src_v.at[j]], bufs.at[p], sem_g).start()

        def gather_wait():
            pltpu.make_async_copy(x_hbm.at[pl.ds(0, B)], bufs.at[0], sem_g).wait()

        def scatter_start(j, p):
            pltpu.async_copy(bufs.at[p], acc.at[dst_v.at[j]], sem_s, add=True)

        def scatter_wait():
            pltpu.make_async_copy(bufs.at[0], acc.at[pl.ds(0, B)], sem_s).wait()

        def count_deg(j):
            ones16 = jnp.ones((16,), jnp.float32)
            for k in range(B // 16):
                idx = dst_v[j, pl.ds(k * 16, 16)]
                plsc.addupdate_scatter(deg_local, [idx], ones16)

        gather_start(0, 0)

        def blk(j, carry):
            gather_wait()
            scatter_start(j, lax.rem(j, 2))
            if with_deg:
                count_deg(j)

            @pl.when(j >= 1)
            def _():
                scatter_wait()

            gather_start(j + 1, lax.rem(j + 1, 2))
            return carry

        lax.fori_loop(0, NBLK - 1, blk, 0)
        gather_wait()
        scatter_start(NBLK - 1, lax.rem(NBLK - 1, 2))
        if with_deg:
            count_deg(NBLK - 1)
        scatter_wait()
        scatter_wait()

        plsc.subcore_barrier()

        pltpu.sync_copy(acc.at[pl.ds(r0, ROWS_PT)], s_hbm.at[pl.ds(r0, ROWS_PT)])
        if with_deg:
            pltpu.sync_copy(deg_local, deg16_hbm.at[s])
        plsc.subcore_barrier()

    @pl.when(c == 0)
    def _():
        phase(xq0, sq0, True)
        phase(xq1, sq1, False)

    @pl.when(c == 1)
    def _():
        phase(xq2, sq2, False)
        phase(xq3, sq3, False)


_sc_agg = functools.partial(
    pl.kernel,
    out_type=(
        jax.ShapeDtypeStruct((ACC_ROWS, Q), jnp.float32),
        jax.ShapeDtypeStruct((ACC_ROWS, Q), jnp.float32),
        jax.ShapeDtypeStruct((ACC_ROWS, Q), jnp.float32),
        jax.ShapeDtypeStruct((ACC_ROWS, Q), jnp.float32),
        jax.ShapeDtypeStruct((TILES, ACC_ROWS), jnp.float32),
    ),
    mesh=plsc.VectorSubcoreMesh(core_axis_name="c", subcore_axis_name="s"),
    compiler_params=pltpu.CompilerParams(use_tc_tiling_on_sc=False, needs_layout_passes=False),
    scratch_types=[
        pltpu.VMEM((NBLK, B), jnp.int32),
        pltpu.VMEM((NBLK, B), jnp.int32),
        pltpu.VMEM((2, B, Q), jnp.float32),
        pltpu.VMEM((ACC_ROWS,), jnp.float32),
        pltpu.VMEM_SHARED((ACC_ROWS, Q), jnp.float32),
        pltpu.SemaphoreType.DMA,
        pltpu.SemaphoreType.DMA,
    ],
)(_sc_body)


M_BLK = 1000


def _tc_body(s0_ref, s1_ref, s2_ref, s3_ref, deg_ref,
             w0_ref, w1_ref, w2_ref, w3_ref, b_ref, out_ref):
    acc = jnp.dot(s0_ref[...], w0_ref[...], preferred_element_type=jnp.float32)
    acc = acc + jnp.dot(s1_ref[...], w1_ref[...], preferred_element_type=jnp.float32)
    acc = acc + jnp.dot(s2_ref[...], w2_ref[...], preferred_element_type=jnp.float32)
    acc = acc + jnp.dot(s3_ref[...], w3_ref[...], preferred_element_type=jnp.float32)
    deg = jnp.maximum(jnp.sum(deg_ref[...], axis=1, keepdims=True), 1.0)
    out_ref[...] = acc / deg + b_ref[...]


_tc_linear = pl.pallas_call(
    _tc_body,
    grid=(N // M_BLK,),
    in_specs=[
        pl.BlockSpec((M_BLK, Q), lambda i: (i, 0)),
        pl.BlockSpec((M_BLK, Q), lambda i: (i, 0)),
        pl.BlockSpec((M_BLK, Q), lambda i: (i, 0)),
        pl.BlockSpec((M_BLK, Q), lambda i: (i, 0)),
        pl.BlockSpec((M_BLK, TILES), lambda i: (i, 0)),
        pl.BlockSpec((Q, D), lambda i: (0, 0)),
        pl.BlockSpec((Q, D), lambda i: (0, 0)),
        pl.BlockSpec((Q, D), lambda i: (0, 0)),
        pl.BlockSpec((Q, D), lambda i: (0, 0)),
        pl.BlockSpec((1, D), lambda i: (0, 0)),
    ],
    out_specs=pl.BlockSpec((M_BLK, D), lambda i: (i, 0)),
    out_shape=jax.ShapeDtypeStruct((N, D), jnp.float32),
)


def kernel(x, edge_index, W, b):
    src = edge_index[0]
    dst = edge_index[1]
    pad = E_PAD - E
    srcp = jnp.concatenate([src, jnp.zeros((pad,), jnp.int32)]).reshape(TILES, NBLK, B)
    dstp = jnp.concatenate([dst, jnp.full((pad,), N, jnp.int32)]).reshape(TILES, NBLK, B)
    xp = jnp.pad(x, ((0, ACC_ROWS - N), (0, 0)))
    s0, s1, s2, s3, deg16 = _sc_agg(xp[:, :Q], xp[:, Q:2 * Q],
                                    xp[:, 2 * Q:3 * Q], xp[:, 3 * Q:],
                                    srcp, dstp)
    wq = [W[:, q * Q:(q + 1) * Q].T for q in range(4)]
    return _tc_linear(s0, s1, s2, s3, deg16.T,
                      wq[0], wq[1], wq[2], wq[3], b.reshape(1, D))


# speedup vs baseline: 1.1576x; 1.1576x over previous
import functools

import jax
import jax.numpy as jnp
from jax import lax
from jax.experimental import pallas as pl
from jax.experimental.pallas import tpu as pltpu
from jax.experimental.pallas import tpu_sc as plsc

N = 10000
E = 160000
D = 256
H = 128
TILES = 16
B = 128
G = 1
NBLK = G * -(-E // (TILES * B * G))
EPT = NBLK * B
E_PAD = TILES * EPT
GB = G * B
ACC_ROWS = 10240
ROWS_PT = ACC_ROWS // TILES


def _sc_body(x0_hbm, x1_hbm, src_hbm, dst_hbm,
             s0_hbm, s1_hbm, deg16_hbm,
             src_v, dst_v, buf, deg_local,
             acc,
             sem_g):
    c = lax.axis_index("c")
    s = lax.axis_index("s")

    pltpu.sync_copy(src_hbm.at[s], src_v)
    pltpu.sync_copy(dst_hbm.at[s], dst_v)

    r0 = s * ROWS_PT

    def zero_deg(i, carry):
        deg_local[pl.ds(i * 16, 16)] = jnp.zeros((16,), jnp.float32)
        return carry

    @pl.when(c == 0)
    def _():
        lax.fori_loop(0, ACC_ROWS // 16, zero_deg, 0)

    @pl.when(c == 0)
    def _():
        pltpu.sync_copy(x0_hbm.at[pl.ds(r0, ROWS_PT)], acc.at[pl.ds(r0, ROWS_PT)])

    @pl.when(c == 1)
    def _():
        pltpu.sync_copy(x1_hbm.at[pl.ds(r0, ROWS_PT)], acc.at[pl.ds(r0, ROWS_PT)])

    plsc.subcore_barrier()

    def edge_loop(x_hbm, with_deg):
        def grp(g, carry):
            gather = pltpu.make_async_copy(
                x_hbm.at[src_v.at[g]], buf, sem_g)
            gather.start()
            if with_deg:
                ones16 = jnp.ones((16,), jnp.float32)
                for k in range(GB // 16):
                    idx = dst_v[g, pl.ds(k * 16, 16)]
                    plsc.addupdate_scatter(deg_local, [idx], ones16)
            gather.wait()
            pltpu.sync_copy(buf, acc.at[dst_v.at[g]], add=True)
            return carry

        lax.fori_loop(0, NBLK // G, grp, 0)

    @pl.when(c == 0)
    def _():
        edge_loop(x0_hbm, True)

    @pl.when(c == 1)
    def _():
        edge_loop(x1_hbm, False)

    plsc.subcore_barrier()

    @pl.when(c == 0)
    def _():
        pltpu.sync_copy(acc.at[pl.ds(r0, ROWS_PT)], s0_hbm.at[pl.ds(r0, ROWS_PT)])
        pltpu.sync_copy(deg_local, deg16_hbm.at[s])

    @pl.when(c == 1)
    def _():
        pltpu.sync_copy(acc.at[pl.ds(r0, ROWS_PT)], s1_hbm.at[pl.ds(r0, ROWS_PT)])


_sc_agg = functools.partial(
    pl.kernel,
    out_type=(
        jax.ShapeDtypeStruct((ACC_ROWS, H), jnp.float32),
        jax.ShapeDtypeStruct((ACC_ROWS, H), jnp.float32),
        jax.ShapeDtypeStruct((TILES, ACC_ROWS), jnp.float32),
    ),
    mesh=plsc.VectorSubcoreMesh(core_axis_name="c", subcore_axis_name="s"),
    compiler_params=pltpu.CompilerParams(use_tc_tiling_on_sc=False,
                                         needs_layout_passes=False),
    scratch_types=[
        pltpu.VMEM((NBLK // G, GB), jnp.int32),
        pltpu.VMEM((NBLK // G, GB), jnp.int32),
        pltpu.VMEM((GB, H), jnp.float32),
        pltpu.VMEM((ACC_ROWS,), jnp.float32),
        pltpu.VMEM_SHARED((ACC_ROWS, H), jnp.float32),
        pltpu.SemaphoreType.DMA,
    ],
)(_sc_body)


M_BLK = 1000


def _tc_body(s0_ref, s1_ref, deg_ref, w0_ref, w1_ref, b_ref, out_ref):
    acc = jnp.dot(s0_ref[...], w0_ref[...], preferred_element_type=jnp.float32)
    acc = acc + jnp.dot(s1_ref[...], w1_ref[...], preferred_element_type=jnp.float32)
    deg = jnp.maximum(jnp.sum(deg_ref[...], axis=1, keepdims=True), 1.0)
    out_ref[...] = acc / deg + b_ref[...]


_tc_linear = pl.pallas_call(
    _tc_body,
    grid=(N // M_BLK,),
    in_specs=[
        pl.BlockSpec((M_BLK, H), lambda i: (i, 0)),
        pl.BlockSpec((M_BLK, H), lambda i: (i, 0)),
        pl.BlockSpec((M_BLK, TILES), lambda i: (i, 0)),
        pl.BlockSpec((H, D), lambda i: (0, 0)),
        pl.BlockSpec((H, D), lambda i: (0, 0)),
        pl.BlockSpec((1, D), lambda i: (0, 0)),
    ],
    out_specs=pl.BlockSpec((M_BLK, D), lambda i: (i, 0)),
    out_shape=jax.ShapeDtypeStruct((N, D), jnp.float32),
)


def kernel(x, edge_index, W, b):
    src = edge_index[0]
    dst = edge_index[1]
    pad = E_PAD - E
    srcp = jnp.concatenate([src, jnp.zeros((pad,), jnp.int32)]).reshape(
        TILES, NBLK // G, GB)
    dstp = jnp.concatenate([dst, jnp.full((pad,), N, jnp.int32)]).reshape(
        TILES, NBLK // G, GB)
    xp = jnp.pad(x, ((0, ACC_ROWS - N), (0, 0)))
    s0, s1, deg16 = _sc_agg(xp[:, :H], xp[:, H:], srcp, dstp)
    w0t = W[:, :H].T
    w1t = W[:, H:].T
    return _tc_linear(s0, s1, deg16.T, w0t, w1t, b.reshape(1, D))
